# baseline (speedup 1.0000x reference)
import functools

import jax
import jax.numpy as jnp
from jax import lax
from jax.experimental import pallas as pl
from jax.experimental.pallas import tpu as pltpu
from jax.experimental.pallas import tpu_sc as plsc

HIDDEN = 64
B, L = 4096, 200
N_TOKENS = B * L
N_CLASSES = 6

_info = plsc.get_sparse_core_info()
NC, NS = _info.num_cores, _info.num_subcores
NW = NC * NS

N_STRIPS = 4
STRIP_B = B // N_STRIPS
CHUNK_ROWS = 4
CHUNK = CHUNK_ROWS * L
GROUPS = CHUNK // 16


def _table_body(emb_ref, w1_ref, b1_ref, w2_ref, b2_ref, out_ref):
    h = jnp.dot(emb_ref[...], w1_ref[...], preferred_element_type=jnp.float32)
    h = h + b1_ref[...]
    h = h * jax.nn.sigmoid(h)
    o = jnp.dot(h, w2_ref[...], preferred_element_type=jnp.float32)
    out_ref[...] = o + b2_ref[...]


def _mlp_table(emb, W1, b1, W2, b2):
    n = emb.shape[0]
    return pl.pallas_call(
        _table_body,
        out_shape=jax.ShapeDtypeStruct((n, HIDDEN), jnp.float32),
    )(emb, W1, b1.reshape(1, HIDDEN), W2, b2.reshape(1, HIDDEN))


def _make_gather(strip):
    rows_per_w = STRIP_B // NW
    b_per_w = rows_per_w * L
    n_chunks = rows_per_w // CHUNK_ROWS
    n_outer = n_chunks // 2
    strip_row0 = strip * STRIP_B
    mesh = plsc.VectorSubcoreMesh(core_axis_name="c", subcore_axis_name="s")

    @functools.partial(
        pl.kernel,
        mesh=mesh,
        out_type=jax.ShapeDtypeStruct((STRIP_B * L * HIDDEN,), jnp.float32),
        scratch_types=[
            pltpu.VMEM((N_CLASSES * HIDDEN,), jnp.float32),
            pltpu.VMEM((CHUNK,), jnp.int32),
            pltpu.VMEM((CHUNK,), jnp.int32),
            pltpu.VMEM((CHUNK * HIDDEN,), jnp.float32),
            pltpu.VMEM((CHUNK * HIDDEN,), jnp.float32),
            pltpu.SemaphoreType.DMA,
            pltpu.SemaphoreType.DMA,
            pltpu.SemaphoreType.DMA,
            pltpu.SemaphoreType.DMA,
        ],
        compiler_params=pltpu.CompilerParams(
            use_tc_tiling_on_sc=False, needs_layout_passes=False),
    )
    def gather_kernel(table_hbm, idx_hbm, out_hbm,
                      tbl_v, idx_a, idx_b, out_a, out_b,
                      si_a, si_b, so_a, so_b):
        wid = lax.axis_index("s") * NC + lax.axis_index("c")
        row_base = strip_row0 + wid * rows_per_w
        base = wid * b_per_w
        pltpu.sync_copy(table_hbm, tbl_v)

        def fire_idx(k, buf, sem):
            for r in range(CHUNK_ROWS):
                pltpu.async_copy(
                    idx_hbm.at[row_base + k * CHUNK_ROWS + r],
                    buf.at[pl.ds(r * L, L)], sem)

        def wait_idx(buf, sem):
            for r in range(CHUNK_ROWS):
                pltpu.make_async_copy(
                    idx_hbm.at[row_base],
                    buf.at[pl.ds(r * L, L)], sem).wait()

        def fire_out(k, buf, sem):
            pltpu.async_copy(
                buf, out_hbm.at[pl.ds((base + k * CHUNK) * HIDDEN, CHUNK * HIDDEN)], sem)

        def wait_out(buf, sem):
            pltpu.make_async_copy(
                buf, out_hbm.at[pl.ds(base * HIDDEN, CHUNK * HIDDEN)], sem).wait()

        def compute(idx_ref, out_ref):
            def grp(g, carry):
                off16 = idx_ref[pl.ds(g * 16, 16)] * HIDDEN
                row0 = g * (16 * HIDDEN)
                for r in range(16):
                    src = off16[r]
                    dst = row0 + r * HIDDEN
                    for c in range(HIDDEN // 16):
                        out_ref[pl.ds(dst + c * 16, 16)] = (
                            tbl_v[pl.ds(src + c * 16, 16)])
                return carry
            lax.fori_loop(0, GROUPS, grp, 0)

        fire_idx(0, idx_a, si_a)
        fire_idx(1, idx_b, si_b)

        def outer(kk, carry):
            for b, (idxv, outv, si, so) in enumerate(
                    ((idx_a, out_a, si_a, so_a), (idx_b, out_b, si_b, so_b))):
                k = kk * 2 + b
                wait_idx(idxv, si)

                @pl.when(kk > 0)
                def _drain():
                    wait_out(outv, so)

                compute(idxv, outv)

                @pl.when(k + 2 < n_chunks)
                def _prefetch():
                    fire_idx(k + 2, idxv, si)

                fire_out(k, outv, so)
            return carry

        lax.fori_loop(0, n_outer, outer, 0)
        wait_out(out_a, so_a)
        wait_out(out_b, so_b)

    return gather_kernel


_gathers = [_make_gather(s) for s in range(N_STRIPS)]

FMT_ROWS = 128
FMT_IN = FMT_ROWS * L * HIDDEN // 128


def _format_body(in_ref, out_ref):
    a = in_ref[...].reshape(FMT_ROWS, L // 2, 128)
    lo = a[:, :, None, :HIDDEN]
    hi = a[:, :, None, HIDDEN:]
    out_ref[...] = jnp.concatenate([lo, hi], axis=2).reshape(
        FMT_ROWS, L, HIDDEN)


def _format(flat):
    v = flat.reshape(N_TOKENS * HIDDEN // 128, 128)
    return pl.pallas_call(
        _format_body,
        grid=(B // FMT_ROWS,),
        in_specs=[pl.BlockSpec((FMT_IN, 128), lambda i: (i, 0))],
        out_specs=pl.BlockSpec((FMT_ROWS, L, HIDDEN), lambda i: (i, 0, 0)),
        out_shape=jax.ShapeDtypeStruct((B, L, HIDDEN), jnp.float32),
    )(v)


def kernel(x, emb, W1, b1, W2, b2):
    table = _mlp_table(emb, W1, b1, W2, b2).reshape(-1)
    xi = x.astype(jnp.int32)
    strips = [g(table, xi) for g in _gathers]
    return _format(jnp.concatenate(strips))


# speedup vs baseline: 1.4649x; 1.4649x over previous
import functools

import jax
import jax.numpy as jnp
from jax import lax
from jax.experimental import pallas as pl
from jax.experimental.pallas import tpu as pltpu
from jax.experimental.pallas import tpu_sc as plsc

HIDDEN = 64
B, L = 4096, 200
N_TOKENS = B * L
N_CLASSES = 6
LANES = 128
OUT_ROWS = N_TOKENS * HIDDEN // LANES

_info = plsc.get_sparse_core_info()
NC, NS = _info.num_cores, _info.num_subcores
NW = NC * NS

CHUNK_ROWS = 4
CHUNK = CHUNK_ROWS * L
CROWS = CHUNK * HIDDEN // LANES
GROUPS = CHUNK // 16


def _table_body(emb_ref, w1_ref, b1_ref, w2_ref, b2_ref, out_ref):
    h = jnp.dot(emb_ref[...], w1_ref[...], preferred_element_type=jnp.float32)
    h = h + b1_ref[...]
    h = h * jax.nn.sigmoid(h)
    o = jnp.dot(h, w2_ref[...], preferred_element_type=jnp.float32)
    out_ref[...] = o + b2_ref[...]


def _mlp_table(emb, W1, b1, W2, b2):
    n = emb.shape[0]
    return pl.pallas_call(
        _table_body,
        out_shape=jax.ShapeDtypeStruct((n, HIDDEN), jnp.float32),
    )(emb, W1, b1.reshape(1, HIDDEN), W2, b2.reshape(1, HIDDEN))


def _make_gather():
    rows_per_w = B // NW
    b_per_w = rows_per_w * L
    orow_per_w = b_per_w * HIDDEN // LANES
    n_chunks = rows_per_w // CHUNK_ROWS
    n_outer = n_chunks // 2
    mesh = plsc.VectorSubcoreMesh(core_axis_name="c", subcore_axis_name="s")

    @functools.partial(
        pl.kernel,
        mesh=mesh,
        out_type=jax.ShapeDtypeStruct((OUT_ROWS, LANES), jnp.float32),
        scratch_types=[
            pltpu.VMEM((N_CLASSES * HIDDEN,), jnp.float32),
            pltpu.VMEM((CHUNK,), jnp.int32),
            pltpu.VMEM((CHUNK,), jnp.int32),
            pltpu.VMEM((CROWS, LANES), jnp.float32),
            pltpu.VMEM((CROWS, LANES), jnp.float32),
            pltpu.SemaphoreType.DMA,
            pltpu.SemaphoreType.DMA,
            pltpu.SemaphoreType.DMA,
            pltpu.SemaphoreType.DMA,
        ],
        compiler_params=pltpu.CompilerParams(
            use_tc_tiling_on_sc=False, needs_layout_passes=False),
    )
    def gather_kernel(table_hbm, idx_hbm, out_hbm,
                      tbl_v, idx_a, idx_b, out_a, out_b,
                      si_a, si_b, so_a, so_b):
        wid = lax.axis_index("s") * NC + lax.axis_index("c")
        row_base = wid * rows_per_w
        obase = wid * orow_per_w
        pltpu.sync_copy(table_hbm, tbl_v)

        def fire_idx(k, buf, sem):
            for r in range(CHUNK_ROWS):
                pltpu.async_copy(
                    idx_hbm.at[row_base + k * CHUNK_ROWS + r],
                    buf.at[pl.ds(r * L, L)], sem)

        def wait_idx(buf, sem):
            for r in range(CHUNK_ROWS):
                pltpu.make_async_copy(
                    idx_hbm.at[row_base],
                    buf.at[pl.ds(r * L, L)], sem).wait()

        def fire_out(k, buf, sem):
            pltpu.async_copy(
                buf, out_hbm.at[pl.ds(obase + k * CROWS, CROWS)], sem)

        def wait_out(buf, sem):
            pltpu.make_async_copy(
                buf, out_hbm.at[pl.ds(obase, CROWS)], sem).wait()

        def compute(idx_ref, out_ref):
            def grp(g, carry):
                off16 = idx_ref[pl.ds(g * 16, 16)] * HIDDEN
                row0 = g * 8
                for p in range(8):
                    se = off16[2 * p]
                    so = off16[2 * p + 1]
                    r = row0 + p
                    for c in range(HIDDEN // 16):
                        out_ref[r, pl.ds(c * 16, 16)] = (
                            tbl_v[pl.ds(se + c * 16, 16)])
                        out_ref[r, pl.ds(HIDDEN + c * 16, 16)] = (
                            tbl_v[pl.ds(so + c * 16, 16)])
                return carry
            lax.fori_loop(0, GROUPS, grp, 0)

        fire_idx(0, idx_a, si_a)
        fire_idx(1, idx_b, si_b)

        def outer(kk, carry):
            for b, (idxv, outv, si, so) in enumerate(
                    ((idx_a, out_a, si_a, so_a), (idx_b, out_b, si_b, so_b))):
                k = kk * 2 + b
                wait_idx(idxv, si)

                @pl.when(kk > 0)
                def _drain():
                    wait_out(outv, so)

                compute(idxv, outv)

                @pl.when(k + 2 < n_chunks)
                def _prefetch():
                    fire_idx(k + 2, idxv, si)

                fire_out(k, outv, so)
            return carry

        lax.fori_loop(0, n_outer, outer, 0)
        wait_out(out_a, so_a)
        wait_out(out_b, so_b)

    return gather_kernel


_gather = _make_gather()


def kernel(x, emb, W1, b1, W2, b2):
    table = _mlp_table(emb, W1, b1, W2, b2)
    out = _gather(table.reshape(-1), x.astype(jnp.int32))
    return out.reshape(B, L, HIDDEN)
